# baseline (speedup 1.0000x reference)
import jax
import jax.numpy as jnp
from jax import lax
from jax.experimental import pallas as pl
from jax.experimental.pallas import tpu as pltpu
from jax.experimental.pallas import tpu_sc as plsc

B = 1024
L = 200
D = 768
N = B * L
NC = 2
NS = 16
NW = NC * NS
LANES = 16
LB = 40
NLB = L // LB
BPW = B // NW


def _sc_body(tok_hbm, seg_hbm, pe_hbm, ti_hbm, sl_hbm, out_hbm,
             pe_v, seg_v,
             ti0, sl0, tok0, ti1, sl1, tok1,
             sem_t0, sem_w0, sem_t1, sem_w1):
    wid = lax.axis_index("s") * NC + lax.axis_index("c")
    b0 = wid * BPW

    pltpu.sync_copy(seg_hbm, seg_v)

    sets = (
        (ti0, sl0, tok0, sem_t0, sem_w0),
        (ti1, sl1, tok1, sem_t1, sem_w1),
    )

    def issue(start, p):
        ti_v, sl_v, tok_v, sem_t, _ = sets[p]
        pltpu.sync_copy(ti_hbm.at[pl.ds(start, LB)], ti_v)
        pltpu.sync_copy(sl_hbm.at[pl.ds(start, LB)], sl_v)
        pltpu.async_copy(tok_hbm.at[ti_v], tok_v, sem_t)

    def wait_gather(p):
        ti_v, _, tok_v, sem_t, _ = sets[p]
        pltpu.make_async_copy(tok_hbm.at[ti_v], tok_v, sem_t).wait()

    def add(p):
        _, sl_v, tok_v, _, _ = sets[p]

        @plsc.parallel_loop(0, LB, unroll=2)
        def _row(r):
            s_splat = plsc.load_gather(sl_v, [jnp.full((LANES,), r, jnp.int32)])
            for c in range(0, D, LANES):
                cvec = c + lax.iota(jnp.int32, LANES)
                seg_slot = plsc.load_gather(seg_v, [s_splat, cvec])
                tok_v.at[r, pl.ds(c, LANES)][...] = (
                    tok_v.at[r, pl.ds(c, LANES)][...]
                    + pe_v.at[r, pl.ds(c, LANES)][...]) + seg_slot

    def start_write(start, p):
        _, _, tok_v, _, sem_w = sets[p]
        pltpu.async_copy(tok_v, out_hbm.at[pl.ds(start, LB)], sem_w)

    def wait_write(start, p):
        _, _, tok_v, _, sem_w = sets[p]
        pltpu.make_async_copy(tok_v, out_hbm.at[pl.ds(start, LB)], sem_w).wait()

    @pl.loop(0, NLB)
    def _segment(lb):
        lboff = lb * LB

        pltpu.sync_copy(pe_hbm.at[pl.ds(lboff, LB)], pe_v)

        def row_start(i):
            return (b0 + i) * L + lboff

        issue(row_start(0), 0)

        @pl.loop(0, BPW, step=2)
        def _tile(i):
            s0 = row_start(i)
            s1 = row_start(i + 1)

            @pl.when(i > 0)
            def _():
                wait_write(s1 - 2 * L, 1)

            issue(s1, 1)
            wait_gather(0)
            add(0)
            start_write(s0, 0)
            wait_gather(1)
            add(1)
            wait_write(s0, 0)

            @pl.when(i + 2 < BPW)
            def _():
                issue(row_start(i + 2), 0)

            start_write(s1, 1)

        wait_write(row_start(BPW - 1), 1)


def kernel(x, segment_label, token_weight, segment_weight, pe):
    ti = x.reshape(N).astype(jnp.int32)
    sl = segment_label.reshape(N).astype(jnp.int32)
    pe_l = pe[0, :L]

    mesh = plsc.VectorSubcoreMesh(core_axis_name="c", subcore_axis_name="s")
    sc = pl.kernel(
        _sc_body,
        out_type=jax.ShapeDtypeStruct((N, D), jnp.float32),
        mesh=mesh,
        compiler_params=pltpu.CompilerParams(needs_layout_passes=False),
        scratch_types=[
            pltpu.VMEM((LB, D), jnp.float32),
            pltpu.VMEM((3, D), jnp.float32),
            pltpu.VMEM((LB,), jnp.int32),
            pltpu.VMEM((LB,), jnp.int32),
            pltpu.VMEM((LB, D), jnp.float32),
            pltpu.VMEM((LB,), jnp.int32),
            pltpu.VMEM((LB,), jnp.int32),
            pltpu.VMEM((LB, D), jnp.float32),
            pltpu.SemaphoreType.DMA,
            pltpu.SemaphoreType.DMA,
            pltpu.SemaphoreType.DMA,
            pltpu.SemaphoreType.DMA,
        ],
    )
    out = sc(token_weight, segment_weight, pe_l, ti, sl)
    return out.reshape(B, L, D)


# speedup vs baseline: 1.2452x; 1.2452x over previous
import jax
import jax.numpy as jnp
from jax import lax
from jax.experimental import pallas as pl
from jax.experimental.pallas import tpu as pltpu
from jax.experimental.pallas import tpu_sc as plsc

B = 1024
L = 200
D = 768
N = B * L
NC = 2
NS = 16
NW = NC * NS
LANES = 16
LB = 40
NLB = L // LB
BPW = B // NW


def _sc_body(tok_hbm, seg_hbm, pe_hbm, ti_hbm, sl_hbm, out_hbm,
             pe_v, seg_v,
             ti0, sl0, sm0, tok0, ti1, sl1, sm1, tok1,
             sem_t0, sem_w0, sem_t1, sem_w1):
    wid = lax.axis_index("s") * NC + lax.axis_index("c")
    b0 = wid * BPW

    pltpu.sync_copy(seg_hbm, seg_v)

    sets = (
        (ti0, sl0, sm0, tok0, sem_t0, sem_w0),
        (ti1, sl1, sm1, tok1, sem_t1, sem_w1),
    )

    def issue(start, p):
        ti_v, sl_v, sl_m, tok_v, sem_t, _ = sets[p]
        pltpu.sync_copy(ti_hbm.at[pl.ds(start, LB)], ti_v)
        pltpu.sync_copy(sl_hbm.at[pl.ds(start, LB)], sl_v.at[pl.ds(0, LB)])
        pltpu.async_copy(tok_hbm.at[ti_v], tok_v, sem_t)

    def wait_gather(p):
        ti_v, _, _, tok_v, sem_t, _ = sets[p]
        pltpu.make_async_copy(tok_hbm.at[ti_v], tok_v, sem_t).wait()

    def add(p):
        _, sl_v, _, tok_v, _, _ = sets[p]

        @plsc.parallel_loop(0, LB, unroll=2)
        def _row(r):
            s = sl_v.at[pl.ds(r, LANES)][...][0]
            for c in range(0, D, LANES):
                tok_v.at[r, pl.ds(c, LANES)][...] = (
                    tok_v.at[r, pl.ds(c, LANES)][...]
                    + pe_v.at[r, pl.ds(c, LANES)][...]
                ) + seg_v.at[s, pl.ds(c, LANES)][...]

    def start_write(start, p):
        _, _, _, tok_v, _, sem_w = sets[p]
        pltpu.async_copy(tok_v, out_hbm.at[pl.ds(start, LB)], sem_w)

    def wait_write(start, p):
        _, _, _, tok_v, _, sem_w = sets[p]
        pltpu.make_async_copy(tok_v, out_hbm.at[pl.ds(start, LB)], sem_w).wait()

    @pl.loop(0, NLB)
    def _segment(lb):
        lboff = lb * LB

        pltpu.sync_copy(pe_hbm.at[pl.ds(lboff, LB)], pe_v)

        def row_start(i):
            return (b0 + i) * L + lboff

        issue(row_start(0), 0)

        @pl.loop(0, BPW, step=2)
        def _tile(i):
            s0 = row_start(i)
            s1 = row_start(i + 1)

            @pl.when(i > 0)
            def _():
                wait_write(s1 - 2 * L, 1)

            issue(s1, 1)
            wait_gather(0)
            add(0)
            start_write(s0, 0)
            wait_gather(1)
            add(1)
            wait_write(s0, 0)

            @pl.when(i + 2 < BPW)
            def _():
                issue(row_start(i + 2), 0)

            start_write(s1, 1)

        wait_write(row_start(BPW - 1), 1)


def kernel(x, segment_label, token_weight, segment_weight, pe):
    ti = x.reshape(N).astype(jnp.int32)
    sl = segment_label.reshape(N).astype(jnp.int32)
    pe_l = pe[0, :L]

    mesh = plsc.VectorSubcoreMesh(core_axis_name="c", subcore_axis_name="s")
    sc = pl.kernel(
        _sc_body,
        out_type=jax.ShapeDtypeStruct((N, D), jnp.float32),
        mesh=mesh,
        scratch_types=[
            pltpu.VMEM((LB, D), jnp.float32),
            pltpu.VMEM((3, D), jnp.float32),
            pltpu.VMEM((LB,), jnp.int32),
            pltpu.VMEM((LB + LANES,), jnp.int32),
            pltpu.SMEM((LB,), jnp.int32),
            pltpu.VMEM((LB, D), jnp.float32),
            pltpu.VMEM((LB,), jnp.int32),
            pltpu.VMEM((LB + LANES,), jnp.int32),
            pltpu.SMEM((LB,), jnp.int32),
            pltpu.VMEM((LB, D), jnp.float32),
            pltpu.SemaphoreType.DMA,
            pltpu.SemaphoreType.DMA,
            pltpu.SemaphoreType.DMA,
            pltpu.SemaphoreType.DMA,
        ],
    )
    out = sc(token_weight, segment_weight, pe_l, ti, sl)
    return out.reshape(B, L, D)
